# baseline (speedup 1.0000x reference)
import math

import numpy as np
import jax
import jax.numpy as jnp
from jax.experimental import pallas as pl
from jax.experimental.pallas import tpu as pltpu

_L = 2048
_N1 = 64
_N2 = 32
_TOPK = int(math.log(_L))
_R = 64


def _split_bf16(a):
    hi = a.astype(np.float32).astype(jnp.bfloat16)
    lo = (a.astype(np.float32) - np.asarray(hi, np.float32)).astype(jnp.bfloat16)
    return jnp.asarray(hi), jnp.asarray(lo)


def _fft_consts():
    n1 = np.arange(_N1)
    n2 = np.arange(_N2)
    w1 = np.exp(-2j * np.pi * np.outer(n1, n1) / _N1)
    w2 = np.exp(-2j * np.pi * np.outer(n2, n2) / _N2)
    tw = np.exp(-2j * np.pi * np.outer(n2, n1) / _L)
    w2c = np.exp(+2j * np.pi * np.outer(n2, n2) / _N2)
    w1c = np.exp(+2j * np.pi * np.outer(n1, n1) / _N1) / _L
    tw2 = np.exp(+2j * np.pi * np.outer(n1, n2) / _L)
    mask = np.ones((_N1, _N2), np.float32)
    mask[0, 0] = 0.0
    mask[0, (_L // 2) // _N1] = 0.0

    w1_ri = np.concatenate([w1.real, w1.imag], axis=1)
    w2_blk = np.block([[w2.real, w2.imag],
                       [-w2.imag, w2.real]])
    w2c_blk = np.block([[w2c.real, w2c.imag],
                        [-w2c.imag, w2c.real]])
    w1c_re = np.concatenate([w1c.real, -w1c.imag], axis=0)
    f32 = lambda a: jnp.asarray(a, jnp.float32)
    return (_split_bf16(w1_ri) + _split_bf16(w2_blk) +
            _split_bf16(w2c_blk) + _split_bf16(w1c_re) +
            (f32(tw.real), f32(tw.imag), f32(tw2.real), f32(tw2.imag),
             f32(mask)))


def _body(q_ref, k_ref, v_ref,
          w1ri_h_ref, w1ri_l_ref, w2blk_h_ref, w2blk_l_ref,
          w2cblk_h_ref, w2cblk_l_ref, w1cre_h_ref, w1cre_l_ref,
          twr_ref, twi_ref, tw2r_ref, tw2i_ref, mask_ref, out_ref):
    w1ri = (w1ri_h_ref[...], w1ri_l_ref[...])
    w2blk = (w2blk_h_ref[...], w2blk_l_ref[...])
    w2cblk = (w2cblk_h_ref[...], w2cblk_l_ref[...])
    w1cre = (w1cre_h_ref[...], w1cre_l_ref[...])
    twr, twi = twr_ref[None], twi_ref[None]
    tw2r, tw2i = tw2r_ref[None], tw2i_ref[None]
    mask = mask_ref[None]

    def dot2d(a, w):
        return jnp.dot(a, w, preferred_element_type=jnp.float32)

    def mm_hi(a, w):
        r, m, c = a.shape
        a2 = a.reshape(r * m, c).astype(jnp.bfloat16)
        return dot2d(a2, w[0]).reshape(r, m, -1)

    def mm_hi_out16(a, w):
        return mm_hi(a, w).astype(jnp.bfloat16)

    def mm3(a, w):
        r, m, c = a.shape
        a2 = a.reshape(r * m, c)
        a_hi = a2.astype(jnp.bfloat16)
        a_lo = (a2 - a_hi.astype(jnp.float32)).astype(jnp.bfloat16)
        out = dot2d(a_hi, w[0]) + (dot2d(a_hi, w[1]) + dot2d(a_lo, w[0]))
        return out.reshape(r, m, -1)

    def swap(a):
        return jnp.swapaxes(a, 1, 2)

    def cat(a, b):
        return jnp.concatenate([a, b], axis=-1)

    def fft_fwd_real(x, mm):
        a = mm(x, w1ri)
        ar, ai = a[:, :, :_N1], a[:, :, _N1:]
        br = ar * twr - ai * twi
        bi = ar * twi + ai * twr
        b = cat(swap(br), swap(bi))
        c = mm(b, w2blk)
        return c[:, :, :_N2], c[:, :, _N2:]

    def ifft_real(xr, xi, mm):
        z = mm(cat(xr, xi), w2cblk)
        zr, zi = z[:, :, :_N2], z[:, :, _N2:]
        yr = zr * tw2r - zi * tw2i
        yi = zr * tw2i + zi * tw2r
        y = cat(swap(yr), swap(yi))
        return mm(y, w1cre)

    q = q_ref[...]
    k = k_ref[...]
    v = v_ref[...].astype(jnp.float32)
    r = q.shape[0]

    qcr, qci = fft_fwd_real(q, mm3)
    kcr, kci = fft_fwd_real(k, mm3)
    pr = (qcr * kcr + qci * kci) * mask
    pi = (qci * kcr - qcr * kci) * mask
    corr = ifft_real(pr, pi, mm3)

    i2 = jax.lax.broadcasted_iota(jnp.int32, (r, _N2, _N1), 1)
    i3 = jax.lax.broadcasted_iota(jnp.int32, (r, _N2, _N1), 2)
    flat = i2 * _N1 + i3
    big = jnp.int32(_L)
    c = corr
    ws, js = [], []
    for _ in range(_TOPK):
        m = jnp.max(jnp.max(c, axis=2, keepdims=True), axis=1, keepdims=True)
        hit = c >= m
        j = jnp.min(jnp.min(jnp.where(hit, flat, big), axis=2, keepdims=True),
                    axis=1, keepdims=True)
        ws.append(m)
        js.append(j)
        c = jnp.where(flat == j, jnp.float32(-1e30), c)

    es = [jnp.exp(w - ws[0]) for w in ws]
    tot = es[0]
    for e in es[1:]:
        tot = tot + e
    inv_tot = 1.0 / tot

    g = jnp.zeros((r, _N2, _N1), jnp.float32)
    for e, j in zip(es, js):
        g = g + jnp.where(flat == j, e * inv_tot, 0.0)

    vcr, vci = fft_fwd_real(v, mm_hi)
    gcr, gci = fft_fwd_real(g, mm_hi)
    sr = vcr * gcr + vci * gci
    si = vci * gcr - vcr * gci
    z = mm_hi(cat(sr, si), w2cblk)
    zr, zi = z[:, :, :_N2], z[:, :, _N2:]
    yr = zr * tw2r - zi * tw2i
    yi = zr * tw2i + zi * tw2r
    y = cat(swap(yr), swap(yi))
    out_ref[...] = mm_hi_out16(y, w1cre)


def kernel(queries, keys, values, attn_mask):
    del attn_mask
    b, l, h, e = queries.shape
    rows = b * h * e

    def tile(x):
        x = jnp.transpose(x, (0, 2, 3, 1)).reshape(rows, _N1, _N2)
        return jnp.transpose(x, (0, 2, 1))

    qt, kt = tile(queries), tile(keys)
    vt = tile(values.astype(jnp.bfloat16))
    consts = _fft_consts()

    cspec = [pl.BlockSpec(cst.shape, lambda i: (0,) * cst.ndim)
             for cst in consts]
    rspec = pl.BlockSpec((_R, _N2, _N1), lambda i: (i, 0, 0))

    out = pl.pallas_call(
        _body,
        grid=(rows // _R,),
        in_specs=[rspec, rspec, rspec] + cspec,
        out_specs=rspec,
        out_shape=jax.ShapeDtypeStruct((rows, _N2, _N1), jnp.bfloat16),
    )(qt, kt, vt, *consts)

    out = jnp.transpose(out, (0, 2, 1)).reshape(b, h, e, l)
    return jnp.transpose(out, (0, 3, 1, 2)).astype(jnp.float32)


# speedup vs baseline: 2.7127x; 2.7127x over previous
import math

import ml_dtypes
import numpy as np
import jax
import jax.numpy as jnp
from jax.experimental import pallas as pl
from jax.experimental.pallas import tpu as pltpu

_L = 2048
_N1 = 128
_N2 = 16
_TOPK = int(math.log(_L))
_R = 64


def _split_bf16(a):
    a = np.asarray(a, np.float32)
    hi = a.astype(ml_dtypes.bfloat16)
    lo = (a - hi.astype(np.float32)).astype(ml_dtypes.bfloat16)
    return jnp.asarray(hi), jnp.asarray(lo)


def _fft_consts():
    n1 = np.arange(_N1)
    n2 = np.arange(_N2)
    w2 = np.exp(-2j * np.pi * np.outer(n2, n2) / _N2)
    w2ri = np.concatenate([w2.real, w2.imag], axis=0)
    tw = np.exp(-2j * np.pi * np.outer(n2, n1) / _L)[:, None, :]
    w1 = np.exp(-2j * np.pi * np.outer(n1, n1) / _N1)
    w1blk = np.block([[w1.real, w1.imag],
                      [-w1.imag, w1.real]])
    w1c = np.exp(+2j * np.pi * np.outer(n1, n1) / _N1) / _L
    w1cblk = np.block([[w1c.real, w1c.imag],
                       [-w1c.imag, w1c.real]])
    tw2 = np.exp(+2j * np.pi * np.outer(n2, n1) / _L)[:, None, :]
    w2c = np.exp(+2j * np.pi * np.outer(n2, n2) / _N2)
    w2cout = np.concatenate([w2c.real, -w2c.imag], axis=1)
    mask = np.ones((_N2, 1, _N1), np.float32)
    mask[0, 0, 0] = 0.0
    mask[0, 0, _L // 2 // _N2] = 0.0
    f32 = lambda a: jnp.asarray(a, jnp.float32)
    return (_split_bf16(w2ri) + _split_bf16(w1blk) + _split_bf16(w1cblk) +
            _split_bf16(w2cout) +
            (f32(tw.real), f32(tw.imag), f32(tw2.real), f32(tw2.imag),
             f32(mask)))


def _body(q_ref, k_ref, v_ref,
          w2ri_h_ref, w2ri_l_ref, w1blk_h_ref, w1blk_l_ref,
          w1cblk_h_ref, w1cblk_l_ref, w2cout_h_ref, w2cout_l_ref,
          twr_ref, twi_ref, tw2r_ref, tw2i_ref, mask_ref, out_ref):
    w2ri = (w2ri_h_ref[...], w2ri_l_ref[...])
    w1blk = (w1blk_h_ref[...], w1blk_l_ref[...])
    w1cblk = (w1cblk_h_ref[...], w1cblk_l_ref[...])
    w2cout = (w2cout_h_ref[...], w2cout_l_ref[...])
    twr, twi = twr_ref[...], twi_ref[...]
    tw2r, tw2i = tw2r_ref[...], tw2i_ref[...]
    mask = mask_ref[...]

    dn = (((1,), (0,)), ((), ()))

    def dg(w, x):
        return jax.lax.dot_general(w, x, dn,
                                   preferred_element_type=jnp.float32)

    def split(x):
        hi = x.astype(jnp.bfloat16)
        lo = (x - hi.astype(jnp.float32)).astype(jnp.bfloat16)
        return hi, lo

    def dg1(w, x):
        return dg(w[0], x.astype(jnp.bfloat16))

    def dg3(w, x):
        x_hi, x_lo = split(x)
        return dg(w[0], x_hi) + (dg(w[1], x_hi) + dg(w[0], x_lo))

    def mm1(a, w):
        s, r, c = a.shape
        a2 = a.reshape(s * r, c).astype(jnp.bfloat16)
        return jnp.dot(a2, w[0],
                       preferred_element_type=jnp.float32).reshape(s, r, -1)

    def mm3(a, w):
        s, r, c = a.shape
        a2 = a.reshape(s * r, c)
        a_hi, a_lo = split(a2)
        f = lambda x, ww: jnp.dot(x, ww, preferred_element_type=jnp.float32)
        out = f(a_hi, w[0]) + (f(a_hi, w[1]) + f(a_lo, w[0]))
        return out.reshape(s, r, -1)

    def cat(a, b):
        return jnp.concatenate([a, b], axis=-1)

    def fft_fwd_real(x, dgx, mmx):
        a = dgx(w2ri, x)
        ar, ai = a[:_N2], a[_N2:]
        br = ar * twr - ai * twi
        bi = ar * twi + ai * twr
        xp = mmx(cat(br, bi), w1blk)
        return xp[:, :, :_N1], xp[:, :, _N1:]

    def ifft_real(xr, xi, dgx, mmx):
        zp = mmx(cat(xr, xi), w1cblk)
        zr, zi = zp[:, :, :_N1], zp[:, :, _N1:]
        yr = zr * tw2r - zi * tw2i
        yi = zr * tw2i + zi * tw2r
        ys = jnp.concatenate([yr, yi], axis=0)
        return dgx(w2cout, ys)

    q = q_ref[...]
    k = k_ref[...]
    v = v_ref[...]
    r = q.shape[1]

    qcr, qci = fft_fwd_real(q, dg3, mm3)
    kcr, kci = fft_fwd_real(k, dg3, mm3)
    pr = (qcr * kcr + qci * kci) * mask
    pi = (qci * kcr - qcr * kci) * mask
    corr = ifft_real(pr, pi, dg3, mm3)

    i0 = jax.lax.broadcasted_iota(jnp.int32, (_N2, r, _N1), 0)
    i2 = jax.lax.broadcasted_iota(jnp.int32, (_N2, r, _N1), 2)
    flat = i0 * _N1 + i2
    big = jnp.int32(_L)
    c = corr
    ws, js = [], []
    for _ in range(_TOPK):
        m = jnp.max(jnp.max(c, axis=0, keepdims=True), axis=2, keepdims=True)
        hit = c >= m
        j = jnp.min(jnp.min(jnp.where(hit, flat, big), axis=0, keepdims=True),
                    axis=2, keepdims=True)
        ws.append(m)
        js.append(j)
        c = jnp.where(flat == j, jnp.float32(-1e30), c)

    es = [jnp.exp(w - ws[0]) for w in ws]
    tot = es[0]
    for e in es[1:]:
        tot = tot + e
    inv_tot = 1.0 / tot

    g = jnp.zeros((_N2, r, _N1), jnp.float32)
    for e, j in zip(es, js):
        g = g + jnp.where(flat == j, e * inv_tot, 0.0)

    vcr, vci = fft_fwd_real(v, dg1, mm1)
    gcr, gci = fft_fwd_real(g, dg1, mm1)
    sr = vcr * gcr + vci * gci
    si = vci * gcr - vcr * gci
    out_ref[...] = ifft_real(sr, si, dg1, mm1)


def kernel(queries, keys, values, attn_mask):
    del attn_mask
    b, l, h, e = queries.shape
    rows = b * h * e

    def tile(x):
        x = jnp.transpose(x, (0, 2, 3, 1)).reshape(rows, _N2, _N1)
        return jnp.transpose(x, (1, 0, 2))

    qt, kt, vt = tile(queries), tile(keys), tile(values)
    consts = _fft_consts()

    cspec = [pl.BlockSpec(cst.shape, lambda i, n=cst.ndim: (0,) * n)
             for cst in consts]
    rspec = pl.BlockSpec((_N2, _R, _N1), lambda i: (0, i, 0))

    out = pl.pallas_call(
        _body,
        grid=(rows // _R,),
        in_specs=[rspec, rspec, rspec] + cspec,
        out_specs=rspec,
        out_shape=jax.ShapeDtypeStruct((_N2, rows, _N1), jnp.float32),
    )(qt, kt, vt, *consts)

    out = jnp.transpose(out, (1, 0, 2)).reshape(b, h, e, l)
    return jnp.transpose(out, (0, 3, 1, 2))


# speedup vs baseline: 2.8871x; 1.0643x over previous
import math

import ml_dtypes
import numpy as np
import jax
import jax.numpy as jnp
from jax.experimental import pallas as pl
from jax.experimental.pallas import tpu as pltpu

_L = 2048
_N1 = 128
_N2 = 16
_TOPK = int(math.log(_L))
_R = 128


def _split_bf16(a):
    a = np.asarray(a, np.float32)
    hi = a.astype(ml_dtypes.bfloat16)
    lo = (a - hi.astype(np.float32)).astype(ml_dtypes.bfloat16)
    return jnp.asarray(hi), jnp.asarray(lo)


def _fft_consts():
    n1 = np.arange(_N1)
    n2 = np.arange(_N2)
    w2 = np.exp(-2j * np.pi * np.outer(n2, n2) / _N2)
    w2ri = np.concatenate([w2.real, w2.imag], axis=0)
    tw = np.exp(-2j * np.pi * np.outer(n2, n1) / _L)[:, None, :]
    w1 = np.exp(-2j * np.pi * np.outer(n1, n1) / _N1)
    w1blk = np.block([[w1.real, w1.imag],
                      [-w1.imag, w1.real]])
    w1c = np.exp(+2j * np.pi * np.outer(n1, n1) / _N1) / _L
    w1cblk = np.block([[w1c.real, w1c.imag],
                       [-w1c.imag, w1c.real]])
    tw2 = np.exp(+2j * np.pi * np.outer(n2, n1) / _L)[:, None, :]
    w2c = np.exp(+2j * np.pi * np.outer(n2, n2) / _N2)
    w2cout = np.concatenate([w2c.real, -w2c.imag], axis=1)
    mask = np.ones((_N2, 1, _N1), np.float32)
    mask[0, 0, 0] = 0.0
    mask[0, 0, _L // 2 // _N2] = 0.0
    f32 = lambda a: jnp.asarray(a, jnp.float32)
    return (_split_bf16(w2ri) + _split_bf16(w1blk) + _split_bf16(w1cblk) +
            _split_bf16(w2cout) +
            (f32(tw.real), f32(tw.imag), f32(tw2.real), f32(tw2.imag),
             f32(mask)))


def _body(q_ref, k_ref, v_ref,
          w2ri_h_ref, w2ri_l_ref, w1blk_h_ref, w1blk_l_ref,
          w1cblk_h_ref, w1cblk_l_ref, w2cout_h_ref, w2cout_l_ref,
          twr_ref, twi_ref, tw2r_ref, tw2i_ref, mask_ref, out_ref):
    w2ri = (w2ri_h_ref[...], w2ri_l_ref[...])
    w1blk = (w1blk_h_ref[...], w1blk_l_ref[...])
    w1cblk = (w1cblk_h_ref[...], w1cblk_l_ref[...])
    w2cout = (w2cout_h_ref[...], w2cout_l_ref[...])
    twr, twi = twr_ref[...], twi_ref[...]
    tw2r, tw2i = tw2r_ref[...], tw2i_ref[...]
    mask = mask_ref[...]

    dn = (((1,), (0,)), ((), ()))

    def dg(w, x):
        return jax.lax.dot_general(w, x, dn,
                                   preferred_element_type=jnp.float32)

    def split(x):
        hi = x.astype(jnp.bfloat16)
        lo = (x - hi.astype(jnp.float32)).astype(jnp.bfloat16)
        return hi, lo

    def dg1(w, x):
        return dg(w[0], x.astype(jnp.bfloat16))

    def dg3(w, x):
        x_hi, x_lo = split(x)
        return dg(w[0], x_hi) + (dg(w[1], x_hi) + dg(w[0], x_lo))

    def mm1(a, w):
        s, r, c = a.shape
        a2 = a.reshape(s * r, c).astype(jnp.bfloat16)
        return jnp.dot(a2, w[0],
                       preferred_element_type=jnp.float32).reshape(s, r, -1)

    def mm3(a, w):
        s, r, c = a.shape
        a2 = a.reshape(s * r, c)
        a_hi, a_lo = split(a2)
        f = lambda x, ww: jnp.dot(x, ww, preferred_element_type=jnp.float32)
        out = f(a_hi, w[0]) + (f(a_hi, w[1]) + f(a_lo, w[0]))
        return out.reshape(s, r, -1)

    def cat(a, b):
        return jnp.concatenate([a, b], axis=-1)

    def fft_fwd_real(x, dgx, mmx):
        a = dgx(w2ri, x)
        ar, ai = a[:_N2], a[_N2:]
        br = ar * twr - ai * twi
        bi = ar * twi + ai * twr
        xp = mmx(cat(br, bi), w1blk)
        return xp[:, :, :_N1], xp[:, :, _N1:]

    def ifft_real(xr, xi, dgx, mmx):
        zp = mmx(cat(xr, xi), w1cblk)
        zr, zi = zp[:, :, :_N1], zp[:, :, _N1:]
        yr = zr * tw2r - zi * tw2i
        yi = zr * tw2i + zi * tw2r
        ys = jnp.concatenate([yr, yi], axis=0)
        return dgx(w2cout, ys)

    q = q_ref[...]
    k = k_ref[...]
    v = v_ref[...]
    r = q.shape[1]

    qcr, qci = fft_fwd_real(q, dg3, mm3)
    kcr, kci = fft_fwd_real(k, dg3, mm3)
    pr = (qcr * kcr + qci * kci) * mask
    pi = (qci * kcr - qcr * kci) * mask
    corr = ifft_real(pr, pi, dg3, mm3)

    i0 = jax.lax.broadcasted_iota(jnp.int32, (_N2, r, _N1), 0)
    i2 = jax.lax.broadcasted_iota(jnp.int32, (_N2, r, _N1), 2)
    flat = i0 * _N1 + i2
    big = jnp.int32(_L)
    c = corr
    ws, js = [], []
    for _ in range(_TOPK):
        m = jnp.max(jnp.max(c, axis=0, keepdims=True), axis=2, keepdims=True)
        hit = c >= m
        j = jnp.min(jnp.min(jnp.where(hit, flat, big), axis=0, keepdims=True),
                    axis=2, keepdims=True)
        ws.append(m)
        js.append(j)
        c = jnp.where(flat == j, jnp.float32(-1e30), c)

    es = [jnp.exp(w - ws[0]) for w in ws]
    tot = es[0]
    for e in es[1:]:
        tot = tot + e
    inv_tot = 1.0 / tot

    g = jnp.zeros((_N2, r, _N1), jnp.float32)
    for e, j in zip(es, js):
        g = g + jnp.where(flat == j, e * inv_tot, 0.0)

    vcr, vci = fft_fwd_real(v, dg1, mm1)
    gcr, gci = fft_fwd_real(g, dg1, mm1)
    sr = vcr * gcr + vci * gci
    si = vci * gcr - vcr * gci
    out_ref[...] = ifft_real(sr, si, dg1, mm1)


def kernel(queries, keys, values, attn_mask):
    del attn_mask
    b, l, h, e = queries.shape
    rows = b * h * e

    def tile(x):
        x = jnp.transpose(x, (0, 2, 3, 1)).reshape(rows, _N2, _N1)
        return jnp.transpose(x, (1, 0, 2))

    qt, kt, vt = tile(queries), tile(keys), tile(values)
    consts = _fft_consts()

    cspec = [pl.BlockSpec(cst.shape, lambda i, n=cst.ndim: (0,) * n)
             for cst in consts]
    rspec = pl.BlockSpec((_N2, _R, _N1), lambda i: (0, i, 0))

    out = pl.pallas_call(
        _body,
        grid=(rows // _R,),
        in_specs=[rspec, rspec, rspec] + cspec,
        out_specs=rspec,
        out_shape=jax.ShapeDtypeStruct((_N2, rows, _N1), jnp.float32),
    )(qt, kt, vt, *consts)

    out = jnp.transpose(out, (1, 0, 2)).reshape(b, h, e, l)
    return jnp.transpose(out, (0, 3, 1, 2))


# speedup vs baseline: 2.9084x; 1.0074x over previous
import math

import ml_dtypes
import numpy as np
import jax
import jax.numpy as jnp
from jax.experimental import pallas as pl
from jax.experimental.pallas import tpu as pltpu

_L = 2048
_N1 = 128
_N2 = 16
_TOPK = int(math.log(_L))
_R = 256


def _split_bf16(a):
    a = np.asarray(a, np.float32)
    hi = a.astype(ml_dtypes.bfloat16)
    lo = (a - hi.astype(np.float32)).astype(ml_dtypes.bfloat16)
    return jnp.asarray(hi), jnp.asarray(lo)


def _fft_consts():
    n1 = np.arange(_N1)
    n2 = np.arange(_N2)
    w2 = np.exp(-2j * np.pi * np.outer(n2, n2) / _N2)
    w2ri = np.concatenate([w2.real, w2.imag], axis=0)
    tw = np.exp(-2j * np.pi * np.outer(n2, n1) / _L)[:, None, :]
    w1 = np.exp(-2j * np.pi * np.outer(n1, n1) / _N1)
    w1blk = np.block([[w1.real, w1.imag],
                      [-w1.imag, w1.real]])
    w1c = np.exp(+2j * np.pi * np.outer(n1, n1) / _N1) / _L
    w1cblk = np.block([[w1c.real, w1c.imag],
                       [-w1c.imag, w1c.real]])
    tw2 = np.exp(+2j * np.pi * np.outer(n2, n1) / _L)[:, None, :]
    w2c = np.exp(+2j * np.pi * np.outer(n2, n2) / _N2)
    w2cout = np.concatenate([w2c.real, -w2c.imag], axis=1)
    mask = np.ones((_N2, 1, _N1), np.float32)
    mask[0, 0, 0] = 0.0
    mask[0, 0, _L // 2 // _N2] = 0.0
    f32 = lambda a: jnp.asarray(a, jnp.float32)
    return (_split_bf16(w2ri) + _split_bf16(w1blk) + _split_bf16(w1cblk) +
            _split_bf16(w2cout) +
            (f32(tw.real), f32(tw.imag), f32(tw2.real), f32(tw2.imag),
             f32(mask)))


def _body(q_ref, k_ref, v_ref,
          w2ri_h_ref, w2ri_l_ref, w1blk_h_ref, w1blk_l_ref,
          w1cblk_h_ref, w1cblk_l_ref, w2cout_h_ref, w2cout_l_ref,
          twr_ref, twi_ref, tw2r_ref, tw2i_ref, mask_ref, out_ref):
    w2ri = (w2ri_h_ref[...], w2ri_l_ref[...])
    w1blk = (w1blk_h_ref[...], w1blk_l_ref[...])
    w1cblk = (w1cblk_h_ref[...], w1cblk_l_ref[...])
    w2cout = (w2cout_h_ref[...], w2cout_l_ref[...])
    twr, twi = twr_ref[...], twi_ref[...]
    tw2r, tw2i = tw2r_ref[...], tw2i_ref[...]
    mask = mask_ref[...]

    dn = (((1,), (0,)), ((), ()))

    def dg(w, x):
        return jax.lax.dot_general(w, x, dn,
                                   preferred_element_type=jnp.float32)

    def split(x):
        hi = x.astype(jnp.bfloat16)
        lo = (x - hi.astype(jnp.float32)).astype(jnp.bfloat16)
        return hi, lo

    def dg1(w, x):
        return dg(w[0], x.astype(jnp.bfloat16))

    def dg3(w, x):
        x_hi, x_lo = split(x)
        return dg(w[0], x_hi) + (dg(w[1], x_hi) + dg(w[0], x_lo))

    def mm1(a, w):
        s, r, c = a.shape
        a2 = a.reshape(s * r, c).astype(jnp.bfloat16)
        return jnp.dot(a2, w[0],
                       preferred_element_type=jnp.float32).reshape(s, r, -1)

    def mm3(a, w):
        s, r, c = a.shape
        a2 = a.reshape(s * r, c)
        a_hi, a_lo = split(a2)
        f = lambda x, ww: jnp.dot(x, ww, preferred_element_type=jnp.float32)
        out = f(a_hi, w[0]) + (f(a_hi, w[1]) + f(a_lo, w[0]))
        return out.reshape(s, r, -1)

    def cat(a, b):
        return jnp.concatenate([a, b], axis=-1)

    def fft_fwd_real(x, dgx, mmx):
        a = dgx(w2ri, x)
        ar, ai = a[:_N2], a[_N2:]
        br = ar * twr - ai * twi
        bi = ar * twi + ai * twr
        xp = mmx(cat(br, bi), w1blk)
        return xp[:, :, :_N1], xp[:, :, _N1:]

    def ifft_real(xr, xi, dgx, mmx):
        zp = mmx(cat(xr, xi), w1cblk)
        zr, zi = zp[:, :, :_N1], zp[:, :, _N1:]
        yr = zr * tw2r - zi * tw2i
        yi = zr * tw2i + zi * tw2r
        ys = jnp.concatenate([yr, yi], axis=0)
        return dgx(w2cout, ys)

    q = q_ref[...]
    k = k_ref[...]
    v = v_ref[...]
    r = q.shape[1]

    qcr, qci = fft_fwd_real(q, dg3, mm3)
    kcr, kci = fft_fwd_real(k, dg3, mm3)
    pr = (qcr * kcr + qci * kci) * mask
    pi = (qci * kcr - qcr * kci) * mask
    corr = ifft_real(pr, pi, dg3, mm3)

    i0 = jax.lax.broadcasted_iota(jnp.int32, (_N2, r, _N1), 0)
    i2 = jax.lax.broadcasted_iota(jnp.int32, (_N2, r, _N1), 2)
    flat = i0 * _N1 + i2
    big = jnp.int32(_L)
    c = corr
    ws, js = [], []
    for _ in range(_TOPK):
        m = jnp.max(jnp.max(c, axis=0, keepdims=True), axis=2, keepdims=True)
        hit = c >= m
        j = jnp.min(jnp.min(jnp.where(hit, flat, big), axis=0, keepdims=True),
                    axis=2, keepdims=True)
        ws.append(m)
        js.append(j)
        c = jnp.where(flat == j, jnp.float32(-1e30), c)

    es = [jnp.exp(w - ws[0]) for w in ws]
    tot = es[0]
    for e in es[1:]:
        tot = tot + e
    inv_tot = 1.0 / tot

    g = jnp.zeros((_N2, r, _N1), jnp.float32)
    for e, j in zip(es, js):
        g = g + jnp.where(flat == j, e * inv_tot, 0.0)

    vcr, vci = fft_fwd_real(v, dg1, mm1)
    gcr, gci = fft_fwd_real(g, dg1, mm1)
    sr = vcr * gcr + vci * gci
    si = vci * gcr - vcr * gci
    out_ref[...] = ifft_real(sr, si, dg1, mm1)


def kernel(queries, keys, values, attn_mask):
    del attn_mask
    b, l, h, e = queries.shape
    rows = b * h * e

    def tile(x):
        x = jnp.transpose(x, (0, 2, 3, 1)).reshape(rows, _N2, _N1)
        return jnp.transpose(x, (1, 0, 2))

    qt, kt, vt = tile(queries), tile(keys), tile(values)
    consts = _fft_consts()

    cspec = [pl.BlockSpec(cst.shape, lambda i, n=cst.ndim: (0,) * n)
             for cst in consts]
    rspec = pl.BlockSpec((_N2, _R, _N1), lambda i: (0, i, 0))

    out = pl.pallas_call(
        _body,
        grid=(rows // _R,),
        in_specs=[rspec, rspec, rspec] + cspec,
        out_specs=rspec,
        out_shape=jax.ShapeDtypeStruct((_N2, rows, _N1), jnp.float32),
    )(qt, kt, vt, *consts)

    out = jnp.transpose(out, (1, 0, 2)).reshape(b, h, e, l)
    return jnp.transpose(out, (0, 3, 1, 2))


# speedup vs baseline: 3.1600x; 1.0865x over previous
import math

import ml_dtypes
import numpy as np
import jax
import jax.numpy as jnp
from jax.experimental import pallas as pl
from jax.experimental.pallas import tpu as pltpu

_L = 2048
_N1 = 128
_N2 = 16
_TOPK = int(math.log(_L))
_R = 256


def _split_bf16(a):
    a = np.asarray(a, np.float32)
    hi = a.astype(ml_dtypes.bfloat16)
    lo = (a - hi.astype(np.float32)).astype(ml_dtypes.bfloat16)
    return jnp.asarray(hi), jnp.asarray(lo)


def _fft_consts():
    n1 = np.arange(_N1)
    n2 = np.arange(_N2)
    w2 = np.exp(-2j * np.pi * np.outer(n2, n2) / _N2)
    w2ri = np.concatenate([w2.real, w2.imag], axis=0)
    tw = np.exp(-2j * np.pi * np.outer(n2, n1) / _L)[:, None, :]
    w1 = np.exp(-2j * np.pi * np.outer(n1, n1) / _N1)
    w1blk = np.block([[w1.real, w1.imag],
                      [-w1.imag, w1.real]])
    w1c = np.exp(+2j * np.pi * np.outer(n1, n1) / _N1) / _L
    w1cblk = np.block([[w1c.real, w1c.imag],
                       [-w1c.imag, w1c.real]])
    tw2 = np.exp(+2j * np.pi * np.outer(n2, n1) / _L)[:, None, :]
    w2c = np.exp(+2j * np.pi * np.outer(n2, n2) / _N2)
    w2cout = np.concatenate([w2c.real, -w2c.imag], axis=1)
    mask = np.ones((_N2, 1, _N1), np.float32)
    mask[0, 0, 0] = 0.0
    mask[0, 0, _L // 2 // _N2] = 0.0
    f32 = lambda a: jnp.asarray(a, jnp.float32)
    return (_split_bf16(w2ri) + _split_bf16(w1blk) + _split_bf16(w1cblk) +
            _split_bf16(w2cout) +
            (f32(tw.real), f32(tw.imag), f32(tw2.real), f32(tw2.imag),
             f32(mask)))


def _body(q_ref, k_ref, v_ref,
          w2ri_h_ref, w2ri_l_ref, w1blk_h_ref, w1blk_l_ref,
          w1cblk_h_ref, w1cblk_l_ref, w2cout_h_ref, w2cout_l_ref,
          twr_ref, twi_ref, tw2r_ref, tw2i_ref, mask_ref, out_ref):
    w2ri = (w2ri_h_ref[...], w2ri_l_ref[...])
    w1blk = (w1blk_h_ref[...], w1blk_l_ref[...])
    w1cblk = (w1cblk_h_ref[...], w1cblk_l_ref[...])
    w2cout = (w2cout_h_ref[...], w2cout_l_ref[...])
    twr, twi = twr_ref[...], twi_ref[...]
    tw2r, tw2i = tw2r_ref[...], tw2i_ref[...]
    mask = mask_ref[...]

    dn = (((1,), (0,)), ((), ()))

    def dg(w, x):
        return jax.lax.dot_general(w, x, dn,
                                   preferred_element_type=jnp.float32)

    def split(x):
        hi = x.astype(jnp.bfloat16)
        lo = (x - hi.astype(jnp.float32)).astype(jnp.bfloat16)
        return hi, lo

    def dg1(w, x):
        return dg(w[0], x.astype(jnp.bfloat16))

    def dg3(w, x):
        wf = w[0].astype(jnp.float32) + w[1].astype(jnp.float32)
        return jax.lax.dot_general(wf, x, dn,
                                   precision=jax.lax.Precision.HIGHEST,
                                   preferred_element_type=jnp.float32)

    def mm1(a, w):
        s, r, c = a.shape
        a2 = a.reshape(s * r, c).astype(jnp.bfloat16)
        return jnp.dot(a2, w[0],
                       preferred_element_type=jnp.float32).reshape(s, r, -1)

    def mm3(a, w):
        s, r, c = a.shape
        a2 = a.reshape(s * r, c)
        a_hi, a_lo = split(a2)
        f = lambda x, ww: jnp.dot(x, ww, preferred_element_type=jnp.float32)
        out = f(a_hi, w[0]) + (f(a_hi, w[1]) + f(a_lo, w[0]))
        return out.reshape(s, r, -1)

    def cat(a, b):
        return jnp.concatenate([a, b], axis=-1)

    def fft_fwd_real(x, dgx, mmx):
        a = dgx(w2ri, x)
        ar, ai = a[:_N2], a[_N2:]
        br = ar * twr - ai * twi
        bi = ar * twi + ai * twr
        xp = mmx(cat(br, bi), w1blk)
        return xp[:, :, :_N1], xp[:, :, _N1:]

    def ifft_real(xr, xi, dgx, mmx):
        zp = mmx(cat(xr, xi), w1cblk)
        zr, zi = zp[:, :, :_N1], zp[:, :, _N1:]
        yr = zr * tw2r - zi * tw2i
        yi = zr * tw2i + zi * tw2r
        ys = jnp.concatenate([yr, yi], axis=0)
        return dgx(w2cout, ys)

    q = q_ref[...]
    k = k_ref[...]
    v = v_ref[...]
    r = q.shape[1]

    qcr, qci = fft_fwd_real(q, dg3, mm3)
    kcr, kci = fft_fwd_real(k, dg3, mm3)
    pr = (qcr * kcr + qci * kci) * mask
    pi = (qci * kcr - qcr * kci) * mask
    corr = ifft_real(pr, pi, dg3, mm3)

    i0 = jax.lax.broadcasted_iota(jnp.int32, (_N2, r, _N1), 0)
    i2 = jax.lax.broadcasted_iota(jnp.int32, (_N2, r, _N1), 2)
    flat = i0 * _N1 + i2
    big = jnp.int32(_L)
    c = corr
    ws, js = [], []
    for _ in range(_TOPK):
        m = jnp.max(jnp.max(c, axis=0, keepdims=True), axis=2, keepdims=True)
        hit = c >= m
        j = jnp.min(jnp.min(jnp.where(hit, flat, big), axis=0, keepdims=True),
                    axis=2, keepdims=True)
        ws.append(m)
        js.append(j)
        c = jnp.where(flat == j, jnp.float32(-1e30), c)

    es = [jnp.exp(w - ws[0]) for w in ws]
    tot = es[0]
    for e in es[1:]:
        tot = tot + e
    inv_tot = 1.0 / tot

    g = jnp.zeros((_N2, r, _N1), jnp.float32)
    for e, j in zip(es, js):
        g = g + jnp.where(flat == j, e * inv_tot, 0.0)

    vcr, vci = fft_fwd_real(v, dg1, mm1)
    gcr, gci = fft_fwd_real(g, dg1, mm1)
    sr = vcr * gcr + vci * gci
    si = vci * gcr - vcr * gci
    out_ref[...] = ifft_real(sr, si, dg1, mm1)


def kernel(queries, keys, values, attn_mask):
    del attn_mask
    b, l, h, e = queries.shape
    rows = b * h * e

    def tile(x):
        x = jnp.transpose(x, (0, 2, 3, 1)).reshape(rows, _N2, _N1)
        return jnp.transpose(x, (1, 0, 2))

    qt, kt, vt = tile(queries), tile(keys), tile(values)
    consts = _fft_consts()

    cspec = [pl.BlockSpec(cst.shape, lambda i, n=cst.ndim: (0,) * n)
             for cst in consts]
    rspec = pl.BlockSpec((_N2, _R, _N1), lambda i: (0, i, 0))

    out = pl.pallas_call(
        _body,
        grid=(rows // _R,),
        in_specs=[rspec, rspec, rspec] + cspec,
        out_specs=rspec,
        out_shape=jax.ShapeDtypeStruct((_N2, rows, _N1), jnp.float32),
    )(qt, kt, vt, *consts)

    out = jnp.transpose(out, (1, 0, 2)).reshape(b, h, e, l)
    return jnp.transpose(out, (0, 3, 1, 2))
